# baseline (speedup 1.0000x reference)
import functools

import jax
import jax.numpy as jnp
from jax import lax
from jax.experimental import pallas as pl
from jax.experimental.pallas import tpu as pltpu
from jax.experimental.pallas import tpu_sc as plsc

NC = 2
NS = 16
L = 16
NW = NC * NS

B = 4096
S = 200
D = 64
DV = D // L

BPW = B // NW

CH0 = 128
CH1 = S - CH0


R_UNROLL = 8

VROW = 1000000 // 2
DW = 2 * D


def _pool_body(x_hbm, emb_hbm, out_hbm, xv, xo, buf0, buf1, out_v, sem0, sem1):
    wid = lax.axis_index("s") * NC + lax.axis_index("c")
    base = wid * BPW

    pltpu.sync_copy(x_hbm.at[pl.ds(base * S, BPW * S)], xv)

    @pl.loop(0, BPW * S // L, unroll=8)
    def _(i):
        v = xv[pl.ds(i * L, L)]
        xo[pl.ds(i * L, L)] = (v & 1) * D
        xv[pl.ds(i * L, L)] = lax.shift_right_logical(v, 1)

    bufs = (buf0, buf1)
    sems = (sem0, sem1)

    def copies(b, k):
        o = b * S
        return (
            pltpu.make_async_copy(
                emb_hbm.at[xv.at[pl.ds(o, CH0)]],
                bufs[k].at[pl.ds(0, CH0)], sems[k]),
            pltpu.make_async_copy(
                emb_hbm.at[xv.at[pl.ds(o + CH0, CH1)]],
                bufs[k].at[pl.ds(CH0, CH1)], sems[k]),
        )

    def issue(b, k):
        for c in copies(b, k):
            c.start()

    issue(0, 0)
    scale = jnp.float32(1.0 / S)

    @pl.loop(0, BPW, step=2)
    def _(b):
        for k in range(2):
            bb = b + k
            nxt = bb + 1

            @pl.when(nxt < BPW)
            def _():
                issue(nxt, (k + 1) % 2)

            for c in copies(bb, k):
                c.wait()

            buf = bufs[k]

            def grp(row0, offv, lanes, accs):
                a = list(accs)
                for rr in lanes:
                    row = row0 + rr
                    off = offv[rr]
                    for j in range(DV):
                        a[j] = a[j] + buf[row, pl.ds(off + j * L, L)]
                return tuple(a)

            def rbody(r, accs):
                offv = xo[pl.ds(bb * S + r * L, L)]
                return grp(r * L, offv, range(L), accs)

            accs = lax.fori_loop(
                0, S // L, rbody,
                tuple(jnp.zeros((L,), jnp.float32) for _ in range(DV)))
            tail = S % L
            if tail:
                offv = xo[pl.ds(bb * S + S - L, L)]
                accs = grp(S - L, offv, range(L - tail, L), accs)
            for j in range(DV):
                out_v[bb, pl.ds(j * L, L)] = accs[j] * scale

    pltpu.sync_copy(out_v, out_hbm.at[pl.ds(base, BPW)])


@functools.lru_cache(maxsize=1)
def _pool():
    return pl.kernel(
        _pool_body,
        out_type=jax.ShapeDtypeStruct((B, D), jnp.float32),
        mesh=plsc.VectorSubcoreMesh(
            core_axis_name="c", subcore_axis_name="s",
            num_cores=NC, num_subcores=NS),
        scratch_types=[
            pltpu.VMEM((BPW * S,), jnp.int32),
            pltpu.VMEM((BPW * S,), jnp.int32),
            pltpu.VMEM((S, DW), jnp.float32),
            pltpu.VMEM((S, DW), jnp.float32),
            pltpu.VMEM((BPW, D), jnp.float32),
            pltpu.SemaphoreType.DMA,
            pltpu.SemaphoreType.DMA,
        ],
    )


def _mlp_body(p_ref, w1t_ref, b1_ref, w2t_ref, b2_ref, w3t_ref, b3_ref, o_ref):
    h = jnp.maximum(
        jnp.dot(p_ref[...], w1t_ref[...], preferred_element_type=jnp.float32)
        + b1_ref[...], 0.0)
    h = jnp.maximum(
        jnp.dot(h, w2t_ref[...], preferred_element_type=jnp.float32)
        + b2_ref[...], 0.0)
    logits = (jnp.dot(h, w3t_ref[...], preferred_element_type=jnp.float32)
              + b3_ref[...])
    m = jnp.max(logits, axis=1, keepdims=True)
    e = jnp.exp(logits - m)
    o_ref[...] = e / jnp.sum(e, axis=1, keepdims=True)


def _mlp(pooled, w1t, b1p, w2t, b2p, w3t, b3p):
    return pl.pallas_call(
        _mlp_body,
        out_shape=jax.ShapeDtypeStruct((B, 128), jnp.float32),
    )(pooled, w1t, b1p, w2t, b2p, w3t, b3p)


def kernel(x, emb, w1, b1, w2, b2, w3, b3):
    x = x.astype(jnp.int32).reshape(B * S)
    pooled = _pool()(x, emb.reshape(VROW, DW))

    w1t = w1.T
    b1p = b1.reshape(1, 128)
    w2t = jnp.zeros((128, 128), jnp.float32).at[:, :32].set(w2.T)
    b2p = jnp.zeros((1, 128), jnp.float32).at[0, :32].set(b2)
    w3t = jnp.zeros((128, 128), jnp.float32).at[:32, :10].set(w3.T)
    b3p = jnp.full((1, 128), -1e30, jnp.float32).at[0, :10].set(b3)

    out = _mlp(pooled, w1t, b1p, w2t, b2p, w3t, b3p)
    return out[:, :10]


# speedup vs baseline: 2.7752x; 2.7752x over previous
import functools

import jax
import jax.numpy as jnp
from jax import lax
from jax.experimental import pallas as pl
from jax.experimental.pallas import tpu as pltpu
from jax.experimental.pallas import tpu_sc as plsc

NC = 2
NS = 16
L = 16
NW = NC * NS

B = 4096
S = 200
D = 64
DV = D // L

BPW = B // NW

CH0 = 128
CH1 = S - CH0


R_UNROLL = 8

VOC = 1000000
VROW2 = 512000
DW = 2 * D
TBLK = 4096
NBLK = VROW2 // TBLK
NBLK_IN_MAX = VOC // TBLK


def _tr_body(a_ref, b_ref, dst_ref):
    ta = jnp.transpose(a_ref[...], (1, 0))
    tb = jnp.transpose(b_ref[...], (1, 0))
    dst_ref[...] = jnp.concatenate([ta, tb], axis=1)


def _transpose(embT):
    return pl.pallas_call(
        _tr_body,
        grid=(NBLK,),
        in_specs=[
            pl.BlockSpec((D, TBLK), lambda i: (0, i)),
            pl.BlockSpec(
                (D, TBLK),
                lambda i: (0, jnp.minimum(NBLK + i, NBLK_IN_MAX))),
        ],
        out_specs=pl.BlockSpec((TBLK, DW), lambda i: (i, 0)),
        out_shape=jax.ShapeDtypeStruct((VROW2, DW), jnp.float32),
    )(embT, embT)


def _pool_body(x_hbm, emb_hbm, out_hbm, xv, buf0, buf1, out_v, sem0, sem1):
    wid = lax.axis_index("s") * NC + lax.axis_index("c")
    base = wid * BPW

    pltpu.sync_copy(x_hbm.at[pl.ds(base * S, BPW * S)], xv)

    @pl.loop(0, BPW * S // L, unroll=8)
    def _(i):
        v = xv[pl.ds(i * L, L)]
        v2 = v + v
        xv[pl.ds(i * L, L)] = jnp.where(v < VROW2, v2, v2 - (2 * VROW2 - 1))

    bufs = (buf0, buf1)
    sems = (sem0, sem1)

    def copies(b, k):
        o = b * S
        return (
            pltpu.make_async_copy(
                emb_hbm.at[xv.at[pl.ds(o, CH0)]],
                bufs[k].at[pl.ds(0, CH0)], sems[k]),
            pltpu.make_async_copy(
                emb_hbm.at[xv.at[pl.ds(o + CH0, CH1)]],
                bufs[k].at[pl.ds(CH0, CH1)], sems[k]),
        )

    def issue(b, k):
        for c in copies(b, k):
            c.start()

    issue(0, 0)
    scale = jnp.float32(1.0 / S)

    @pl.loop(0, BPW, step=2)
    def _(b):
        for k in range(2):
            bb = b + k
            nxt = bb + 1

            @pl.when(nxt < BPW)
            def _():
                issue(nxt, (k + 1) % 2)

            for c in copies(bb, k):
                c.wait()

            buf = bufs[k]

            def rbody(r, accs):
                a = list(accs)
                for rr in range(R_UNROLL):
                    row = r * R_UNROLL + rr
                    for j in range(DV):
                        a[j] = a[j] + buf[row, pl.ds(j * L, L)]
                return tuple(a)

            accs = lax.fori_loop(
                0, S // R_UNROLL, rbody,
                tuple(jnp.zeros((L,), jnp.float32) for _ in range(DV)))
            for j in range(DV):
                out_v[bb, pl.ds(j * L, L)] = accs[j] * scale

    pltpu.sync_copy(out_v, out_hbm.at[pl.ds(base, BPW)])


@functools.lru_cache(maxsize=1)
def _pool():
    return pl.kernel(
        _pool_body,
        out_type=jax.ShapeDtypeStruct((B, D), jnp.float32),
        mesh=plsc.VectorSubcoreMesh(
            core_axis_name="c", subcore_axis_name="s",
            num_cores=NC, num_subcores=NS),
        scratch_types=[
            pltpu.VMEM((BPW * S,), jnp.int32),
            pltpu.VMEM((S, D), jnp.float32),
            pltpu.VMEM((S, D), jnp.float32),
            pltpu.VMEM((BPW, D), jnp.float32),
            pltpu.SemaphoreType.DMA,
            pltpu.SemaphoreType.DMA,
        ],
        compiler_params=pltpu.CompilerParams(use_tc_tiling_on_sc=False),
    )


def _mlp_body(p_ref, w1t_ref, b1_ref, w2t_ref, b2_ref, w3t_ref, b3_ref, o_ref):
    h = jnp.maximum(
        jnp.dot(p_ref[...], w1t_ref[...], preferred_element_type=jnp.float32)
        + b1_ref[...], 0.0)
    h = jnp.maximum(
        jnp.dot(h, w2t_ref[...], preferred_element_type=jnp.float32)
        + b2_ref[...], 0.0)
    logits = (jnp.dot(h, w3t_ref[...], preferred_element_type=jnp.float32)
              + b3_ref[...])
    m = jnp.max(logits, axis=1, keepdims=True)
    e = jnp.exp(logits - m)
    o_ref[...] = e / jnp.sum(e, axis=1, keepdims=True)


def _mlp(pooled, w1t, b1p, w2t, b2p, w3t, b3p):
    return pl.pallas_call(
        _mlp_body,
        out_shape=jax.ShapeDtypeStruct((B, 128), jnp.float32),
    )(pooled, w1t, b1p, w2t, b2p, w3t, b3p)


def kernel(x, emb, w1, b1, w2, b2, w3, b3):
    x = x.astype(jnp.int32).reshape(B * S)
    emb2 = _transpose(emb.T)
    pooled = _pool()(x, emb2.reshape(2 * VROW2, D))

    w1t = w1.T
    b1p = b1.reshape(1, 128)
    w2t = jnp.zeros((128, 128), jnp.float32).at[:, :32].set(w2.T)
    b2p = jnp.zeros((1, 128), jnp.float32).at[0, :32].set(b2)
    w3t = jnp.zeros((128, 128), jnp.float32).at[:32, :10].set(w3.T)
    b3p = jnp.full((1, 128), -1e30, jnp.float32).at[0, :10].set(b3)

    out = _mlp(pooled, w1t, b1p, w2t, b2p, w3t, b3p)
    return out[:, :10]


# speedup vs baseline: 3.0238x; 1.0896x over previous
import functools

import jax
import jax.numpy as jnp
from jax import lax
from jax.experimental import pallas as pl
from jax.experimental.pallas import tpu as pltpu
from jax.experimental.pallas import tpu_sc as plsc

NC = 2
NS = 16
L = 16
NW = NC * NS

B = 4096
S = 200
D = 64
DV = D // L

BPW = B // NW

CH0 = 128
CH1 = S - CH0


R_UNROLL = 8

VOC = 1000000
VROW2 = 516096
DW = 2 * D
TBLK = 8192
NBLK = VROW2 // TBLK
NBLK_IN_MAX = VOC // TBLK


def _tr_body(a_ref, b_ref, dst_ref):
    ta = jnp.transpose(a_ref[...], (1, 0))
    tb = jnp.transpose(b_ref[...], (1, 0))
    dst_ref[...] = jnp.concatenate([ta, tb], axis=1)


def _transpose(embT):
    return pl.pallas_call(
        _tr_body,
        grid=(NBLK,),
        in_specs=[
            pl.BlockSpec((D, TBLK), lambda i: (0, i)),
            pl.BlockSpec(
                (D, TBLK),
                lambda i: (0, jnp.minimum(NBLK + i, NBLK_IN_MAX))),
        ],
        out_specs=pl.BlockSpec((TBLK, DW), lambda i: (i, 0)),
        out_shape=jax.ShapeDtypeStruct((VROW2, DW), jnp.float32),
    )(embT, embT)


def _pool_body(x_hbm, emb_hbm, out_hbm, xv, buf0, buf1, out_v, sem0, sem1):
    wid = lax.axis_index("s") * NC + lax.axis_index("c")
    base = wid * BPW

    pltpu.sync_copy(x_hbm.at[pl.ds(base * S, BPW * S)], xv)

    @pl.loop(0, BPW * S // L, unroll=8)
    def _(i):
        v = xv[pl.ds(i * L, L)]
        v2 = v + v
        xv[pl.ds(i * L, L)] = jnp.where(v < VROW2, v2, v2 - (2 * VROW2 - 1))

    bufs = (buf0, buf1)
    sems = (sem0, sem1)

    def copies(b, k):
        o = b * S
        return (
            pltpu.make_async_copy(
                emb_hbm.at[xv.at[pl.ds(o, CH0)]],
                bufs[k].at[pl.ds(0, CH0)], sems[k]),
            pltpu.make_async_copy(
                emb_hbm.at[xv.at[pl.ds(o + CH0, CH1)]],
                bufs[k].at[pl.ds(CH0, CH1)], sems[k]),
        )

    def issue(b, k):
        for c in copies(b, k):
            c.start()

    issue(0, 0)
    scale = jnp.float32(1.0 / S)

    @pl.loop(0, BPW, step=2)
    def _(b):
        for k in range(2):
            bb = b + k
            nxt = bb + 1

            @pl.when(nxt < BPW)
            def _():
                issue(nxt, (k + 1) % 2)

            for c in copies(bb, k):
                c.wait()

            buf = bufs[k]

            def rbody(r, accs):
                a = list(accs)
                for rr in range(R_UNROLL):
                    row = r * R_UNROLL + rr
                    for j in range(DV):
                        a[j] = a[j] + buf[row, pl.ds(j * L, L)]
                return tuple(a)

            accs = lax.fori_loop(
                0, S // R_UNROLL, rbody,
                tuple(jnp.zeros((L,), jnp.float32) for _ in range(DV)))
            for j in range(DV):
                out_v[bb, pl.ds(j * L, L)] = accs[j] * scale

    pltpu.sync_copy(out_v, out_hbm.at[pl.ds(base, BPW)])


@functools.lru_cache(maxsize=1)
def _pool():
    return pl.kernel(
        _pool_body,
        out_type=jax.ShapeDtypeStruct((B, D), jnp.float32),
        mesh=plsc.VectorSubcoreMesh(
            core_axis_name="c", subcore_axis_name="s",
            num_cores=NC, num_subcores=NS),
        scratch_types=[
            pltpu.VMEM((BPW * S,), jnp.int32),
            pltpu.VMEM((S, D), jnp.float32),
            pltpu.VMEM((S, D), jnp.float32),
            pltpu.VMEM((BPW, D), jnp.float32),
            pltpu.SemaphoreType.DMA,
            pltpu.SemaphoreType.DMA,
        ],
        compiler_params=pltpu.CompilerParams(use_tc_tiling_on_sc=False),
    )


def _mlp_body(p_ref, w1t_ref, b1_ref, w2t_ref, b2_ref, w3t_ref, b3_ref, o_ref):
    h = jnp.maximum(
        jnp.dot(p_ref[...], w1t_ref[...], preferred_element_type=jnp.float32)
        + b1_ref[...], 0.0)
    h = jnp.maximum(
        jnp.dot(h, w2t_ref[...], preferred_element_type=jnp.float32)
        + b2_ref[...], 0.0)
    logits = (jnp.dot(h, w3t_ref[...], preferred_element_type=jnp.float32)
              + b3_ref[...])
    m = jnp.max(logits, axis=1, keepdims=True)
    e = jnp.exp(logits - m)
    o_ref[...] = e / jnp.sum(e, axis=1, keepdims=True)


def _mlp(pooled, w1t, b1p, w2t, b2p, w3t, b3p):
    return pl.pallas_call(
        _mlp_body,
        out_shape=jax.ShapeDtypeStruct((B, 128), jnp.float32),
    )(pooled, w1t, b1p, w2t, b2p, w3t, b3p)


def kernel(x, emb, w1, b1, w2, b2, w3, b3):
    x = x.astype(jnp.int32).reshape(B * S)
    emb2 = _transpose(emb.T)
    pooled = _pool()(x, emb2.reshape(2 * VROW2, D))

    w1t = w1.T
    b1p = b1.reshape(1, 128)
    w2t = jnp.zeros((128, 128), jnp.float32).at[:, :32].set(w2.T)
    b2p = jnp.zeros((1, 128), jnp.float32).at[0, :32].set(b2)
    w3t = jnp.zeros((128, 128), jnp.float32).at[:32, :10].set(w3.T)
    b3p = jnp.full((1, 128), -1e30, jnp.float32).at[0, :10].set(b3)

    out = _mlp(pooled, w1t, b1p, w2t, b2p, w3t, b3p)
    return out[:, :10]
